# baseline (speedup 1.0000x reference)
import functools

import jax
import jax.numpy as jnp
from jax import lax
from jax.experimental import pallas as pl
from jax.experimental.pallas import tpu as pltpu
from jax.experimental.pallas import tpu_sc as plsc

N = 10000
E = 320000
G = 64
NPAD = 10240
LANE = 128
EPAD = 327680
CHUNKS = EPAD // LANE
ROWS_PER_TILE = NPAD // 16
BLK = 2048

f32 = jnp.float32
i32 = jnp.int32


def _zero_block(ref):
    def body(r, _):
        for k in range(8):
            ref[r, pl.ds(k * 16, 16)] = jnp.zeros((16,), f32)
        return 0
    lax.fori_loop(0, 128, body, 0)


def _deg_body(edges, degp, dstv, onesv, drainv, degsh):
    c = lax.axis_index("c")
    s = lax.axis_index("s")
    w = c * 16 + s
    nch = EPAD // 32 // LANE

    def init(k, _):
        drainv[pl.ds(k * 16, 16)] = jnp.zeros((16,), f32)
        return 0
    lax.fori_loop(0, ROWS_PER_TILE // 16, init, 0)
    for k in range(LANE // 16):
        onesv[pl.ds(k * 16, 16)] = jnp.ones((16,), f32)
    pltpu.sync_copy(drainv, degsh.at[pl.ds(s * ROWS_PER_TILE, ROWS_PER_TILE)])
    plsc.subcore_barrier()

    pltpu.sync_copy(edges.at[1, pl.ds(w * nch, nch)], dstv)

    def body(j, _):
        pltpu.sync_copy(onesv, degsh.at[dstv.at[j]], add=True)
        return 0
    lax.fori_loop(0, nch, body, 0)
    plsc.subcore_barrier()

    pltpu.sync_copy(degsh.at[pl.ds(s * ROWS_PER_TILE, ROWS_PER_TILE)], drainv)
    pltpu.sync_copy(
        drainv, degp.at[pl.ds(c * NPAD + s * ROWS_PER_TILE, ROWS_PER_TILE)])


STAGE = 40


def _mp_loop(copy_stage, table, srcv, dstv, rbuf, accsh, gs0, gs1, ss0, ss1,
             nstages):
    npairs = STAGE // 2

    def stage_body(st, _):
        copy_stage(st)
        pltpu.async_copy(table.at[srcv.at[0]], rbuf.at[0], gs0)
        pltpu.async_copy(table.at[srcv.at[1]], rbuf.at[1], gs1)

        def pair(j2, _):
            j0 = j2 * 2
            j1 = j0 + 1
            pltpu.make_async_copy(table.at[srcv.at[j0]], rbuf.at[0],
                                  gs0).wait()
            pltpu.async_copy(rbuf.at[0], accsh.at[dstv.at[j0]], ss0,
                             add=True)
            pltpu.make_async_copy(table.at[srcv.at[j1]], rbuf.at[1],
                                  gs1).wait()
            pltpu.async_copy(rbuf.at[1], accsh.at[dstv.at[j1]], ss1,
                             add=True)

            @pl.when(j2 < npairs - 1)
            def _():
                pltpu.make_async_copy(rbuf.at[0], accsh.at[dstv.at[j0]],
                                      ss0).wait()
                pltpu.async_copy(table.at[srcv.at[j0 + 2]], rbuf.at[0], gs0)
                pltpu.make_async_copy(rbuf.at[1], accsh.at[dstv.at[j1]],
                                      ss1).wait()
                pltpu.async_copy(table.at[srcv.at[j1 + 2]], rbuf.at[1], gs1)
            return 0
        lax.fori_loop(0, npairs, pair, 0)
        pltpu.make_async_copy(rbuf.at[0], accsh.at[dstv.at[0]], ss0).wait()
        pltpu.make_async_copy(rbuf.at[1], accsh.at[dstv.at[1]], ss1).wait()
        return 0
    lax.fori_loop(0, nstages, stage_body, 0)


def _mp_prologue(rbuf, accsh, s):
    _zero_block(rbuf.at[0])
    for q in range(ROWS_PER_TILE // LANE):
        pltpu.sync_copy(
            rbuf.at[0], accsh.at[pl.ds(s * ROWS_PER_TILE + q * LANE, LANE)])
    plsc.subcore_barrier()


def _mp_drain(rbuf, accsh, out, c, s):
    plsc.subcore_barrier()
    for q in range(ROWS_PER_TILE // LANE):
        rows = pl.ds(s * ROWS_PER_TILE + q * LANE, LANE)
        pltpu.sync_copy(accsh.at[rows], rbuf.at[0])
        pltpu.sync_copy(rbuf.at[0], out.at[c, rows])


def _make_mp_body(nch, per_core_edges):
    def body(edges, src_hi, table, acc, srcv, dstv, rbuf, accsh, gs0, gs1,
             ss0, ss1):
        c = lax.axis_index("c")
        s = lax.axis_index("s")
        base = ((c * 16 + s) if per_core_edges else s) * nch
        _mp_prologue(rbuf, accsh, s)

        def copy_stage(st):
            rows = pl.ds(base + st * STAGE, STAGE)

            @pl.when(c == 0)
            def _():
                pltpu.sync_copy(edges.at[0, rows], srcv)

            @pl.when(c == 1)
            def _():
                pltpu.sync_copy(src_hi.at[rows], srcv)

            pltpu.sync_copy(edges.at[1, rows], dstv)

        _mp_loop(copy_stage, table, srcv, dstv, rbuf, accsh, gs0, gs1,
                 ss0, ss1, nch // STAGE)
        _mp_drain(rbuf, accsh, acc, c, s)
    return body


_mp1_body = _make_mp_body(EPAD // 32 // LANE, True)
_mp2_body = _make_mp_body(EPAD // 16 // LANE, False)


def _pool_body(h2, bnds, gout, bndv, cbuf, maccv):
    c = lax.axis_index("c")
    s = lax.axis_index("s")
    t = c * 16 + s
    g0 = t * 2
    pltpu.sync_copy(bnds, bndv)
    bvec = bndv[0, pl.ds(g0, 16)]
    c0, c1, c2 = bvec[0], bvec[1], bvec[2]

    for gi in range(2):
        start = c0 if gi == 0 else c1
        end = c1 if gi == 0 else c2
        minit = tuple(jnp.full((16,), -jnp.inf, f32) for _ in range(16))
        nchunks = (end - start + 63) // 64

        def chunk_body(ch, m):
            pos = start + ch * 64
            cstart = jnp.minimum((pos // 8) * 8, N - 72)
            off = pos - cstart
            pltpu.sync_copy(h2.at[pl.ds(cstart, 72)], cbuf)
            nv = jnp.minimum(64, end - pos)

            def row_body(r, m):
                row = off + r
                return tuple(
                    jnp.maximum(m[cc], cbuf[row, pl.ds(cc * 16, 16)])
                    for cc in range(16))
            return lax.fori_loop(0, nv, row_body, m)

        m = lax.fori_loop(0, nchunks, chunk_body, minit)
        for cc in range(16):
            maccv[pl.ds(cc * 16, 16)] = m[cc]
        pltpu.sync_copy(maccv, gout.at[pl.ds((g0 + gi) * 256, 256)])


@functools.cache
def _sc_kernels():
    mesh = plsc.VectorSubcoreMesh(core_axis_name="c", subcore_axis_name="s")
    deg = pl.kernel(
        _deg_body,
        out_type=jax.ShapeDtypeStruct((2 * NPAD,), f32),
        scratch_types=[
            pltpu.VMEM((EPAD // 32 // LANE, LANE), i32),
            pltpu.VMEM((LANE,), f32),
            pltpu.VMEM((ROWS_PER_TILE,), f32),
            pltpu.VMEM_SHARED((NPAD,), f32),
        ],
        mesh=mesh,
    )
    mp_scratch = [
        pltpu.VMEM((STAGE, LANE), i32),
        pltpu.VMEM((STAGE, LANE), i32),
        pltpu.VMEM((2, LANE, 128), f32),
        pltpu.VMEM_SHARED((NPAD, 128), f32),
        pltpu.SemaphoreType.DMA,
        pltpu.SemaphoreType.DMA,
        pltpu.SemaphoreType.DMA,
        pltpu.SemaphoreType.DMA,
    ]
    mp1 = pl.kernel(
        _mp1_body,
        out_type=jax.ShapeDtypeStruct((2, NPAD, 128), f32),
        scratch_types=list(mp_scratch),
        mesh=mesh,
    )
    mp2 = pl.kernel(
        _mp2_body,
        out_type=jax.ShapeDtypeStruct((2, NPAD, 128), f32),
        scratch_types=list(mp_scratch),
        mesh=mesh,
    )
    pool = pl.kernel(
        _pool_body,
        out_type=jax.ShapeDtypeStruct((G * 256,), f32),
        scratch_types=[
            pltpu.VMEM((1, 128), i32),
            pltpu.VMEM((72, 256), f32),
            pltpu.VMEM((256,), f32),
        ],
        mesh=mesh,
    )
    return deg, mp1, mp2, pool


def _tc1_body(x_ref, w_ref, deg_ref, batch_ref, hs_ref, dinv_ref, bnd_ref):
    d = deg_ref[0, :] + deg_ref[1, :] + 1.0
    dinv = lax.rsqrt(d)
    dinv_ref[0, :] = dinv
    h = jnp.dot(x_ref[...], w_ref[...], preferred_element_type=f32)
    hs = h * dinv[:, None]
    hs_ref[0] = hs
    hs_ref[1] = hs

    @pl.when(pl.program_id(0) == 0)
    def _():
        b = batch_ref[0, :][:, None]
        gids = lax.broadcasted_iota(i32, (1, 128), 1)
        bnd_ref[...] = jnp.sum((b < gids).astype(i32), axis=0, keepdims=True)


def _tc2_body(acc_ref, hs1_ref, dinv_ref, b1_ref, w2_ref, out_ref):
    dinv = dinv_ref[0, :]
    z = dinv[:, None] * (acc_ref[0] + acc_ref[1] + hs1_ref[0])
    z = jnp.maximum(z + b1_ref[0, :][None, :], 0.0)
    h = jnp.dot(z, w2_ref[...], preferred_element_type=f32) * dinv[:, None]
    out_ref[0] = h[:, :128]
    out_ref[1] = h[:, 128:]


def _tc3_body(acc_ref, hs2_ref, dinv_ref, b2_ref, out_ref):
    dinv = dinv_ref[0, :]
    for half in range(2):
        v = dinv[:, None] * (acc_ref[half] + hs2_ref[half])
        v = jnp.maximum(v + b2_ref[half, :][None, :], 0.0)
        out_ref[:, half * 128:(half + 1) * 128] = v


def _tc4_body(g_ref, w1_ref, b1_ref, w2_ref, b2_ref, out_ref):
    t = jnp.dot(g_ref[...], w1_ref[...], preferred_element_type=f32)
    t = t + b1_ref[0, :][None, :]
    o = jnp.dot(t, w2_ref[...], preferred_element_type=f32)
    out_ref[...] = o + b2_ref[0, :][None, :]


def kernel(x, edge_index, batch, W1, b1, W2, b2, fc1_W, fc1_b, fc2_W, fc2_b):
    pad = EPAD - E
    src = jnp.concatenate([edge_index[0], jnp.zeros((pad,), i32)])
    dst = jnp.concatenate(
        [edge_index[1], N + (jnp.arange(pad, dtype=i32) % (NPAD - N))])
    edges = jnp.stack([src, dst]).reshape(2, CHUNKS, LANE)
    src_hi = (src + N).reshape(CHUNKS, LANE)

    _deg_kernel, _mp1_kernel, _mp2_kernel, _pool_kernel = _sc_kernels()
    degp = _deg_kernel(edges).reshape(2, NPAD)

    batch_pad = jnp.concatenate(
        [batch, jnp.full((NPAD - N,), G, i32)]).reshape(1, NPAD)

    grid = NPAD // BLK
    hs1, dinv, bnds = pl.pallas_call(
        _tc1_body,
        grid=(grid,),
        in_specs=[
            pl.BlockSpec((BLK, 128), lambda i: (i, 0)),
            pl.BlockSpec((128, 128), lambda i: (0, 0)),
            pl.BlockSpec((2, BLK), lambda i: (0, i)),
            pl.BlockSpec((1, NPAD), lambda i: (0, 0)),
        ],
        out_specs=[
            pl.BlockSpec((2, BLK, 128), lambda i: (0, i, 0)),
            pl.BlockSpec((1, BLK), lambda i: (0, i)),
            pl.BlockSpec((1, 128), lambda i: (0, 0)),
        ],
        out_shape=[
            jax.ShapeDtypeStruct((2, N, 128), f32),
            jax.ShapeDtypeStruct((1, NPAD), f32),
            jax.ShapeDtypeStruct((1, 128), i32),
        ],
    )(x, W1, degp, batch_pad)

    hs1f = hs1.reshape(2 * N, 128)
    acc1 = _mp1_kernel(edges, src_hi, hs1f)

    hs2 = pl.pallas_call(
        _tc2_body,
        grid=(grid,),
        in_specs=[
            pl.BlockSpec((2, BLK, 128), lambda i: (0, i, 0)),
            pl.BlockSpec((1, BLK, 128), lambda i: (0, i, 0)),
            pl.BlockSpec((1, BLK), lambda i: (0, i)),
            pl.BlockSpec((1, 128), lambda i: (0, 0)),
            pl.BlockSpec((128, 256), lambda i: (0, 0)),
        ],
        out_specs=pl.BlockSpec((2, BLK, 128), lambda i: (0, i, 0)),
        out_shape=jax.ShapeDtypeStruct((2, N, 128), f32),
    )(acc1, hs1, dinv, b1.reshape(1, 128), W2)

    hs2f = hs2.reshape(2 * N, 128)
    acc2 = _mp2_kernel(edges, src_hi, hs2f)

    h2 = pl.pallas_call(
        _tc3_body,
        grid=(grid,),
        in_specs=[
            pl.BlockSpec((2, BLK, 128), lambda i: (0, i, 0)),
            pl.BlockSpec((2, BLK, 128), lambda i: (0, i, 0)),
            pl.BlockSpec((1, BLK), lambda i: (0, i)),
            pl.BlockSpec((2, 128), lambda i: (0, 0)),
        ],
        out_specs=pl.BlockSpec((BLK, 256), lambda i: (i, 0)),
        out_shape=jax.ShapeDtypeStruct((N, 256), f32),
    )(acc2, hs2, dinv, b2.reshape(2, 128))

    g = _pool_kernel(h2, bnds).reshape(G, 256)

    out = pl.pallas_call(
        _tc4_body,
        in_specs=[
            pl.BlockSpec((G, 256), lambda: (0, 0)),
            pl.BlockSpec((256, 512), lambda: (0, 0)),
            pl.BlockSpec((1, 512), lambda: (0, 0)),
            pl.BlockSpec((512, 128), lambda: (0, 0)),
            pl.BlockSpec((1, 128), lambda: (0, 0)),
        ],
        out_specs=pl.BlockSpec((G, 128), lambda: (0, 0)),
        out_shape=jax.ShapeDtypeStruct((G, 128), f32),
    )(g, fc1_W, fc1_b.reshape(1, 512), fc2_W, fc2_b.reshape(1, 128))
    return out


# speedup vs baseline: 2.2475x; 2.2475x over previous
import functools

import jax
import jax.numpy as jnp
from jax import lax
from jax.experimental import pallas as pl
from jax.experimental.pallas import tpu as pltpu
from jax.experimental.pallas import tpu_sc as plsc

N = 10000
E = 320000
G = 64
NPAD = 10240
LANE = 128
EPAD = 327680
CHUNKS = EPAD // LANE
ROWS_PER_TILE = NPAD // 16
BLK = 2048

f32 = jnp.float32
i32 = jnp.int32


def _zero_block(ref):
    def body(r, _):
        for k in range(8):
            ref[r, pl.ds(k * 16, 16)] = jnp.zeros((16,), f32)
        return 0
    lax.fori_loop(0, 128, body, 0)


def _deg_body(edges, degp, dstv, onesv, drainv, degsh):
    c = lax.axis_index("c")
    s = lax.axis_index("s")
    w = c * 16 + s
    nch = EPAD // 32 // LANE

    def init(k, _):
        drainv[pl.ds(k * 16, 16)] = jnp.zeros((16,), f32)
        return 0
    lax.fori_loop(0, ROWS_PER_TILE // 16, init, 0)
    for k in range(LANE // 16):
        onesv[pl.ds(k * 16, 16)] = jnp.ones((16,), f32)
    pltpu.sync_copy(drainv, degsh.at[pl.ds(s * ROWS_PER_TILE, ROWS_PER_TILE)])
    plsc.subcore_barrier()

    pltpu.sync_copy(edges.at[1, pl.ds(w * nch, nch)], dstv)

    def body(j, _):
        pltpu.sync_copy(onesv, degsh.at[dstv.at[j]], add=True)
        return 0
    lax.fori_loop(0, nch, body, 0)
    plsc.subcore_barrier()

    pltpu.sync_copy(degsh.at[pl.ds(s * ROWS_PER_TILE, ROWS_PER_TILE)], drainv)
    pltpu.sync_copy(
        drainv, degp.at[pl.ds(c * NPAD + s * ROWS_PER_TILE, ROWS_PER_TILE)])


STAGE = 40


def _mp_loop(copy_stage, table, srcv, dstv, rbuf, accsh, gs0, gs1, ss0, ss1,
             nstages):
    npairs = STAGE // 2

    def stage_body(st, _):
        copy_stage(st)
        pltpu.async_copy(table.at[srcv.at[0]], rbuf.at[0], gs0)
        pltpu.async_copy(table.at[srcv.at[1]], rbuf.at[1], gs1)

        def pair(j2, _):
            j0 = j2 * 2
            j1 = j0 + 1
            pltpu.make_async_copy(table.at[srcv.at[j0]], rbuf.at[0],
                                  gs0).wait()
            pltpu.async_copy(rbuf.at[0], accsh.at[dstv.at[j0]], ss0,
                             add=True)
            pltpu.make_async_copy(table.at[srcv.at[j1]], rbuf.at[1],
                                  gs1).wait()
            pltpu.async_copy(rbuf.at[1], accsh.at[dstv.at[j1]], ss1,
                             add=True)

            @pl.when(j2 < npairs - 1)
            def _():
                pltpu.make_async_copy(rbuf.at[0], accsh.at[dstv.at[j0]],
                                      ss0).wait()
                pltpu.async_copy(table.at[srcv.at[j0 + 2]], rbuf.at[0], gs0)
                pltpu.make_async_copy(rbuf.at[1], accsh.at[dstv.at[j1]],
                                      ss1).wait()
                pltpu.async_copy(table.at[srcv.at[j1 + 2]], rbuf.at[1], gs1)
            return 0
        lax.fori_loop(0, npairs, pair, 0)
        pltpu.make_async_copy(rbuf.at[0], accsh.at[dstv.at[0]], ss0).wait()
        pltpu.make_async_copy(rbuf.at[1], accsh.at[dstv.at[1]], ss1).wait()
        return 0
    lax.fori_loop(0, nstages, stage_body, 0)


def _mp_prologue(rbuf, accsh, s):
    _zero_block(rbuf.at[0])
    for q in range(ROWS_PER_TILE // LANE):
        pltpu.sync_copy(
            rbuf.at[0], accsh.at[pl.ds(s * ROWS_PER_TILE + q * LANE, LANE)])
    plsc.subcore_barrier()


def _mp_drain(rbuf, accsh, out, c, s):
    plsc.subcore_barrier()
    for q in range(ROWS_PER_TILE // LANE):
        rows = pl.ds(s * ROWS_PER_TILE + q * LANE, LANE)
        pltpu.sync_copy(accsh.at[rows], rbuf.at[0])
        pltpu.sync_copy(rbuf.at[0], out.at[c, rows])


def _make_mp_body(nch, per_core_edges):
    def body(edges, src_hi, table, acc, srcv, dstv, rbuf, accsh, gs0, gs1,
             ss0, ss1):
        c = lax.axis_index("c")
        s = lax.axis_index("s")
        base = ((c * 16 + s) if per_core_edges else s) * nch
        _mp_prologue(rbuf, accsh, s)

        def copy_stage(st):
            rows = pl.ds(base + st * STAGE, STAGE)

            @pl.when(c == 0)
            def _():
                pltpu.sync_copy(edges.at[0, rows], srcv)

            @pl.when(c == 1)
            def _():
                pltpu.sync_copy(src_hi.at[rows], srcv)

            pltpu.sync_copy(edges.at[1, rows], dstv)

        _mp_loop(copy_stage, table, srcv, dstv, rbuf, accsh, gs0, gs1,
                 ss0, ss1, nch // STAGE)
        _mp_drain(rbuf, accsh, acc, c, s)
    return body


_mp1_body = _make_mp_body(EPAD // 32 // LANE, True)
_mp2_body = _make_mp_body(EPAD // 16 // LANE, False)


def _pool_body(h2, bnds, gout, bndv, cbuf, maccv):
    c = lax.axis_index("c")
    s = lax.axis_index("s")
    t = c * 16 + s
    g0 = t * 2
    pltpu.sync_copy(bnds, bndv)
    bvec = bndv[0, pl.ds(g0, 16)]
    c0, c1, c2 = bvec[0], bvec[1], bvec[2]

    for gi in range(2):
        start = c0 if gi == 0 else c1
        end = c1 if gi == 0 else c2
        minit = tuple(jnp.full((16,), -jnp.inf, f32) for _ in range(16))
        nchunks = (end - start + 63) // 64

        def chunk_body(ch, m):
            pos = start + ch * 64
            cstart = jnp.minimum((pos // 8) * 8, N - 72)
            off = pos - cstart
            pltpu.sync_copy(h2.at[pl.ds(cstart, 72)], cbuf)
            nv = jnp.minimum(64, end - pos)

            def row_body(r, m):
                row = off + r
                return tuple(
                    jnp.maximum(m[cc], cbuf[row, pl.ds(cc * 16, 16)])
                    for cc in range(16))
            return lax.fori_loop(0, nv, row_body, m)

        m = lax.fori_loop(0, nchunks, chunk_body, minit)
        for cc in range(16):
            maccv[pl.ds(cc * 16, 16)] = m[cc]
        pltpu.sync_copy(maccv, gout.at[pl.ds((g0 + gi) * 256, 256)])


@functools.cache
def _sc_kernels():
    mesh = plsc.VectorSubcoreMesh(core_axis_name="c", subcore_axis_name="s")
    deg = pl.kernel(
        _deg_body,
        out_type=jax.ShapeDtypeStruct((2 * NPAD,), f32),
        scratch_types=[
            pltpu.VMEM((EPAD // 32 // LANE, LANE), i32),
            pltpu.VMEM((LANE,), f32),
            pltpu.VMEM((ROWS_PER_TILE,), f32),
            pltpu.VMEM_SHARED((NPAD,), f32),
        ],
        mesh=mesh,
    )
    mp_scratch = [
        pltpu.VMEM((STAGE, LANE), i32),
        pltpu.VMEM((STAGE, LANE), i32),
        pltpu.VMEM((2, LANE, 128), f32),
        pltpu.VMEM_SHARED((NPAD, 128), f32),
        pltpu.SemaphoreType.DMA,
        pltpu.SemaphoreType.DMA,
        pltpu.SemaphoreType.DMA,
        pltpu.SemaphoreType.DMA,
    ]
    mp1 = pl.kernel(
        _mp1_body,
        out_type=jax.ShapeDtypeStruct((2, NPAD, 128), f32),
        scratch_types=list(mp_scratch),
        mesh=mesh,
    )
    mp2 = pl.kernel(
        _mp2_body,
        out_type=jax.ShapeDtypeStruct((2, NPAD, 128), f32),
        scratch_types=list(mp_scratch),
        mesh=mesh,
    )
    pool = pl.kernel(
        _pool_body,
        out_type=jax.ShapeDtypeStruct((G * 256,), f32),
        scratch_types=[
            pltpu.VMEM((1, 128), i32),
            pltpu.VMEM((72, 256), f32),
            pltpu.VMEM((256,), f32),
        ],
        mesh=mesh,
    )
    return deg, mp1, mp2, pool


def _tc1_body(x_ref, w_ref, deg_ref, batch_ref, hs_ref, dinv_ref, bnd_ref):
    d = deg_ref[0, :] + deg_ref[1, :] + 1.0
    dinv = lax.rsqrt(d)
    dinv_ref[0, :] = dinv
    h = jnp.dot(x_ref[...], w_ref[...], preferred_element_type=f32)
    hs = h * dinv[:, None]
    hs_ref[0] = hs
    hs_ref[1] = hs

    @pl.when(pl.program_id(0) == 0)
    def _():
        b = batch_ref[0, :][:, None]
        gids = lax.broadcasted_iota(i32, (1, 128), 1)
        bnd_ref[...] = jnp.sum((b < gids).astype(i32), axis=0, keepdims=True)


def _tc2_body(acc_ref, hs1_ref, dinv_ref, b1_ref, w2_ref, out_ref):
    dinv = dinv_ref[0, :]
    z = dinv[:, None] * (acc_ref[0] + acc_ref[1] + hs1_ref[0])
    z = jnp.maximum(z + b1_ref[0, :][None, :], 0.0)
    h = jnp.dot(z, w2_ref[...], preferred_element_type=f32) * dinv[:, None]
    out_ref[0] = h[:, :128]
    out_ref[1] = h[:, 128:]


def _tc3_body(acc_ref, hs2_ref, dinv_ref, b2_ref, out_ref):
    dinv = dinv_ref[0, :]
    for half in range(2):
        v = dinv[:, None] * (acc_ref[half] + hs2_ref[half])
        v = jnp.maximum(v + b2_ref[half, :][None, :], 0.0)
        out_ref[:, half * 128:(half + 1) * 128] = v


def _tc4_body(g_ref, w1_ref, b1_ref, w2_ref, b2_ref, out_ref):
    t = jnp.dot(g_ref[...], w1_ref[...], preferred_element_type=f32)
    t = t + b1_ref[0, :][None, :]
    o = jnp.dot(t, w2_ref[...], preferred_element_type=f32)
    out_ref[...] = o + b2_ref[0, :][None, :]


def kernel(x, edge_index, batch, W1, b1, W2, b2, fc1_W, fc1_b, fc2_W, fc2_b):
    pad = EPAD - E
    src = jnp.concatenate(
        [edge_index[0], jnp.arange(pad, dtype=i32) % N])
    dst = jnp.concatenate(
        [edge_index[1], N + (jnp.arange(pad, dtype=i32) % (NPAD - N))])
    edges = jnp.stack([src, dst]).reshape(2, CHUNKS, LANE)
    src_hi = (src + N).reshape(CHUNKS, LANE)

    _deg_kernel, _mp1_kernel, _mp2_kernel, _pool_kernel = _sc_kernels()
    degp = _deg_kernel(edges).reshape(2, NPAD)

    batch_pad = jnp.concatenate(
        [batch, jnp.full((NPAD - N,), G, i32)]).reshape(1, NPAD)

    grid = NPAD // BLK
    hs1, dinv, bnds = pl.pallas_call(
        _tc1_body,
        grid=(grid,),
        in_specs=[
            pl.BlockSpec((BLK, 128), lambda i: (i, 0)),
            pl.BlockSpec((128, 128), lambda i: (0, 0)),
            pl.BlockSpec((2, BLK), lambda i: (0, i)),
            pl.BlockSpec((1, NPAD), lambda i: (0, 0)),
        ],
        out_specs=[
            pl.BlockSpec((2, BLK, 128), lambda i: (0, i, 0)),
            pl.BlockSpec((1, BLK), lambda i: (0, i)),
            pl.BlockSpec((1, 128), lambda i: (0, 0)),
        ],
        out_shape=[
            jax.ShapeDtypeStruct((2, N, 128), f32),
            jax.ShapeDtypeStruct((1, NPAD), f32),
            jax.ShapeDtypeStruct((1, 128), i32),
        ],
    )(x, W1, degp, batch_pad)

    hs1f = hs1.reshape(2 * N, 128)
    acc1 = _mp1_kernel(edges, src_hi, hs1f)

    hs2 = pl.pallas_call(
        _tc2_body,
        grid=(grid,),
        in_specs=[
            pl.BlockSpec((2, BLK, 128), lambda i: (0, i, 0)),
            pl.BlockSpec((1, BLK, 128), lambda i: (0, i, 0)),
            pl.BlockSpec((1, BLK), lambda i: (0, i)),
            pl.BlockSpec((1, 128), lambda i: (0, 0)),
            pl.BlockSpec((128, 256), lambda i: (0, 0)),
        ],
        out_specs=pl.BlockSpec((2, BLK, 128), lambda i: (0, i, 0)),
        out_shape=jax.ShapeDtypeStruct((2, N, 128), f32),
    )(acc1, hs1, dinv, b1.reshape(1, 128), W2)

    hs2f = hs2.reshape(2 * N, 128)
    acc2 = _mp2_kernel(edges, src_hi, hs2f)

    h2 = pl.pallas_call(
        _tc3_body,
        grid=(grid,),
        in_specs=[
            pl.BlockSpec((2, BLK, 128), lambda i: (0, i, 0)),
            pl.BlockSpec((2, BLK, 128), lambda i: (0, i, 0)),
            pl.BlockSpec((1, BLK), lambda i: (0, i)),
            pl.BlockSpec((2, 128), lambda i: (0, 0)),
        ],
        out_specs=pl.BlockSpec((BLK, 256), lambda i: (i, 0)),
        out_shape=jax.ShapeDtypeStruct((N, 256), f32),
    )(acc2, hs2, dinv, b2.reshape(2, 128))

    g = _pool_kernel(h2, bnds).reshape(G, 256)

    out = pl.pallas_call(
        _tc4_body,
        in_specs=[
            pl.BlockSpec((G, 256), lambda: (0, 0)),
            pl.BlockSpec((256, 512), lambda: (0, 0)),
            pl.BlockSpec((1, 512), lambda: (0, 0)),
            pl.BlockSpec((512, 128), lambda: (0, 0)),
            pl.BlockSpec((1, 128), lambda: (0, 0)),
        ],
        out_specs=pl.BlockSpec((G, 128), lambda: (0, 0)),
        out_shape=jax.ShapeDtypeStruct((G, 128), f32),
    )(g, fc1_W, fc1_b.reshape(1, 512), fc2_W, fc2_b.reshape(1, 128))
    return out


# speedup vs baseline: 2.2479x; 1.0002x over previous
import functools

import jax
import jax.numpy as jnp
from jax import lax
from jax.experimental import pallas as pl
from jax.experimental.pallas import tpu as pltpu
from jax.experimental.pallas import tpu_sc as plsc

N = 10000
E = 320000
G = 64
NPAD = 10240
LANE = 128
EPAD = 327680
CHUNKS = EPAD // LANE
ROWS_PER_TILE = NPAD // 16
BLK = 2048

f32 = jnp.float32
i32 = jnp.int32


def _zero_block(ref):
    def body(r, _):
        for k in range(8):
            ref[r, pl.ds(k * 16, 16)] = jnp.zeros((16,), f32)
        return 0
    lax.fori_loop(0, 128, body, 0)


def _deg_body(edges, degp, dstv, onesv, drainv, degsh):
    c = lax.axis_index("c")
    s = lax.axis_index("s")
    w = c * 16 + s
    nch = EPAD // 32 // LANE

    def init(k, _):
        drainv[pl.ds(k * 16, 16)] = jnp.zeros((16,), f32)
        return 0
    lax.fori_loop(0, ROWS_PER_TILE // 16, init, 0)
    for k in range(LANE // 16):
        onesv[pl.ds(k * 16, 16)] = jnp.ones((16,), f32)
    pltpu.sync_copy(drainv, degsh.at[pl.ds(s * ROWS_PER_TILE, ROWS_PER_TILE)])
    plsc.subcore_barrier()

    pltpu.sync_copy(edges.at[1, pl.ds(w * nch, nch)], dstv)

    def body(j, _):
        pltpu.sync_copy(onesv, degsh.at[dstv.at[j]], add=True)
        return 0
    lax.fori_loop(0, nch, body, 0)
    plsc.subcore_barrier()

    pltpu.sync_copy(degsh.at[pl.ds(s * ROWS_PER_TILE, ROWS_PER_TILE)], drainv)
    pltpu.sync_copy(
        drainv, degp.at[pl.ds(c * NPAD + s * ROWS_PER_TILE, ROWS_PER_TILE)])


STAGE = 40


def _mp_loop(copy_stage, table, srcv, dstv, rbuf, accsh, gs0, gs1, ss0, ss1,
             nstages):
    npairs = STAGE // 2

    def stage_body(st, _):
        copy_stage(st)
        pltpu.async_copy(table.at[srcv.at[0]], rbuf.at[0], gs0)
        pltpu.async_copy(table.at[srcv.at[1]], rbuf.at[1], gs1)

        def pair(j2, _):
            j0 = j2 * 2
            j1 = j0 + 1
            pltpu.make_async_copy(table.at[srcv.at[j0]], rbuf.at[0],
                                  gs0).wait()
            pltpu.async_copy(rbuf.at[0], accsh.at[dstv.at[j0]], ss0,
                             add=True)
            pltpu.make_async_copy(table.at[srcv.at[j1]], rbuf.at[1],
                                  gs1).wait()
            pltpu.async_copy(rbuf.at[1], accsh.at[dstv.at[j1]], ss1,
                             add=True)

            @pl.when(j2 < npairs - 1)
            def _():
                pltpu.make_async_copy(rbuf.at[0], accsh.at[dstv.at[j0]],
                                      ss0).wait()
                pltpu.async_copy(table.at[srcv.at[j0 + 2]], rbuf.at[0], gs0)
                pltpu.make_async_copy(rbuf.at[1], accsh.at[dstv.at[j1]],
                                      ss1).wait()
                pltpu.async_copy(table.at[srcv.at[j1 + 2]], rbuf.at[1], gs1)
            return 0
        lax.fori_loop(0, npairs, pair, 0)
        pltpu.make_async_copy(rbuf.at[0], accsh.at[dstv.at[0]], ss0).wait()
        pltpu.make_async_copy(rbuf.at[1], accsh.at[dstv.at[1]], ss1).wait()
        return 0
    lax.fori_loop(0, nstages, stage_body, 0)


def _mp_prologue(rbuf, accsh, s):
    _zero_block(rbuf.at[0])
    for q in range(ROWS_PER_TILE // LANE):
        pltpu.sync_copy(
            rbuf.at[0], accsh.at[pl.ds(s * ROWS_PER_TILE + q * LANE, LANE)])
    plsc.subcore_barrier()


def _mp_drain(rbuf, accsh, out, c, s):
    plsc.subcore_barrier()
    for q in range(ROWS_PER_TILE // LANE):
        rows = pl.ds(s * ROWS_PER_TILE + q * LANE, LANE)
        pltpu.sync_copy(accsh.at[rows], rbuf.at[0])
        pltpu.sync_copy(rbuf.at[0], out.at[c, rows])


def _make_mp_body(nch, per_core_edges):
    def body(edges, src_hi, table, acc, srcv, dstv, rbuf, accsh, gs0, gs1,
             ss0, ss1):
        c = lax.axis_index("c")
        s = lax.axis_index("s")
        base = ((c * 16 + s) if per_core_edges else s) * nch
        _mp_prologue(rbuf, accsh, s)

        def copy_stage(st):
            rows = pl.ds(base + st * STAGE, STAGE)

            @pl.when(c == 0)
            def _():
                pltpu.sync_copy(edges.at[0, rows], srcv)

            @pl.when(c == 1)
            def _():
                pltpu.sync_copy(src_hi.at[rows], srcv)

            pltpu.sync_copy(edges.at[1, rows], dstv)

        _mp_loop(copy_stage, table, srcv, dstv, rbuf, accsh, gs0, gs1,
                 ss0, ss1, nch // STAGE)
        _mp_drain(rbuf, accsh, acc, c, s)
    return body


_mp1_body = _make_mp_body(EPAD // 32 // LANE, True)
_mp2_body = _make_mp_body(EPAD // 16 // LANE, False)


def _pool_body(h2, bnds, gout, bndv, cbuf, maccv):
    c = lax.axis_index("c")
    s = lax.axis_index("s")
    t = c * 16 + s
    g0 = t * 2
    pltpu.sync_copy(bnds, bndv)
    bvec = bndv[0, pl.ds(g0, 16)]
    c0, c1, c2 = bvec[0], bvec[1], bvec[2]

    for gi in range(2):
        start = c0 if gi == 0 else c1
        end = c1 if gi == 0 else c2
        minit = tuple(jnp.full((16,), -jnp.inf, f32) for _ in range(16))
        nchunks = (end - start + 63) // 64

        def chunk_body(ch, m):
            pos = start + ch * 64
            cstart = jnp.minimum((pos // 8) * 8, N - 72)
            off = pos - cstart
            pltpu.sync_copy(h2.at[pl.ds(cstart, 72)], cbuf)
            nv = jnp.minimum(64, end - pos)

            def row_body(r, m):
                row = off + r
                return tuple(
                    jnp.maximum(m[cc], cbuf[row, pl.ds(cc * 16, 16)])
                    for cc in range(16))
            return lax.fori_loop(0, nv, row_body, m)

        m = lax.fori_loop(0, nchunks, chunk_body, minit)
        for cc in range(16):
            maccv[pl.ds(cc * 16, 16)] = m[cc]
        pltpu.sync_copy(maccv, gout.at[pl.ds((g0 + gi) * 256, 256)])


@functools.cache
def _sc_kernels():
    mesh = plsc.VectorSubcoreMesh(core_axis_name="c", subcore_axis_name="s")
    deg = pl.kernel(
        _deg_body,
        out_type=jax.ShapeDtypeStruct((2 * NPAD,), f32),
        scratch_types=[
            pltpu.VMEM((EPAD // 32 // LANE, LANE), i32),
            pltpu.VMEM((LANE,), f32),
            pltpu.VMEM((ROWS_PER_TILE,), f32),
            pltpu.VMEM_SHARED((NPAD,), f32),
        ],
        mesh=mesh,
    )
    mp_scratch = [
        pltpu.VMEM((STAGE, LANE), i32),
        pltpu.VMEM((STAGE, LANE), i32),
        pltpu.VMEM((2, LANE, 128), f32),
        pltpu.VMEM_SHARED((NPAD, 128), f32),
        pltpu.SemaphoreType.DMA,
        pltpu.SemaphoreType.DMA,
        pltpu.SemaphoreType.DMA,
        pltpu.SemaphoreType.DMA,
    ]
    mp1 = pl.kernel(
        _mp1_body,
        out_type=jax.ShapeDtypeStruct((2, NPAD, 128), f32),
        scratch_types=list(mp_scratch),
        mesh=mesh,
    )
    mp2 = pl.kernel(
        _mp2_body,
        out_type=jax.ShapeDtypeStruct((2, NPAD, 128), f32),
        scratch_types=list(mp_scratch),
        mesh=mesh,
    )
    pool = pl.kernel(
        _pool_body,
        out_type=jax.ShapeDtypeStruct((G * 256,), f32),
        scratch_types=[
            pltpu.VMEM((1, 128), i32),
            pltpu.VMEM((72, 256), f32),
            pltpu.VMEM((256,), f32),
        ],
        mesh=mesh,
    )
    return deg, mp1, mp2, pool


def _tc1_body(x_ref, w_ref, deg_ref, batch_ref, hs_ref, dinv_ref, bnd_ref):
    d = deg_ref[0, :] + deg_ref[1, :] + 1.0
    dinv = lax.rsqrt(d)
    dinv_ref[0, :] = dinv
    h = jnp.dot(x_ref[...], w_ref[...], preferred_element_type=f32)
    hs_ref[0] = h * dinv[:, None]

    @pl.when(pl.program_id(0) == 0)
    def _():
        b = batch_ref[0, :][:, None]
        gids = lax.broadcasted_iota(i32, (1, 128), 1)
        bnd_ref[...] = jnp.sum((b < gids).astype(i32), axis=0, keepdims=True)


def _tc2_body(acc_ref, hs1_ref, dinv_ref, b1_ref, w2_ref, out_ref):
    dinv = dinv_ref[0, :]
    z = dinv[:, None] * (acc_ref[0] + acc_ref[1] + hs1_ref[0])
    z = jnp.maximum(z + b1_ref[0, :][None, :], 0.0)
    h = jnp.dot(z, w2_ref[...], preferred_element_type=f32) * dinv[:, None]
    out_ref[0] = h[:, :128]
    out_ref[1] = h[:, 128:]


def _tc3_body(acc_ref, hs2_ref, dinv_ref, b2_ref, out_ref):
    dinv = dinv_ref[0, :]
    for half in range(2):
        v = dinv[:, None] * (acc_ref[half] + hs2_ref[half])
        v = jnp.maximum(v + b2_ref[half, :][None, :], 0.0)
        out_ref[:, half * 128:(half + 1) * 128] = v


def _tc4_body(g_ref, w1_ref, b1_ref, w2_ref, b2_ref, out_ref):
    t = jnp.dot(g_ref[...], w1_ref[...], preferred_element_type=f32)
    t = t + b1_ref[0, :][None, :]
    o = jnp.dot(t, w2_ref[...], preferred_element_type=f32)
    out_ref[...] = o + b2_ref[0, :][None, :]


def kernel(x, edge_index, batch, W1, b1, W2, b2, fc1_W, fc1_b, fc2_W, fc2_b):
    pad = EPAD - E
    src = jnp.concatenate(
        [edge_index[0], jnp.arange(pad, dtype=i32) % N])
    dst = jnp.concatenate(
        [edge_index[1], N + (jnp.arange(pad, dtype=i32) % (NPAD - N))])
    edges = jnp.stack([src, dst]).reshape(2, CHUNKS, LANE)
    src_hi = (src + N).reshape(CHUNKS, LANE)

    _deg_kernel, _mp1_kernel, _mp2_kernel, _pool_kernel = _sc_kernels()
    degp = _deg_kernel(edges).reshape(2, NPAD)

    batch_pad = jnp.concatenate(
        [batch, jnp.full((NPAD - N,), G, i32)]).reshape(1, NPAD)

    grid = NPAD // BLK
    hs1, dinv, bnds = pl.pallas_call(
        _tc1_body,
        grid=(grid,),
        in_specs=[
            pl.BlockSpec((BLK, 128), lambda i: (i, 0)),
            pl.BlockSpec((128, 128), lambda i: (0, 0)),
            pl.BlockSpec((2, BLK), lambda i: (0, i)),
            pl.BlockSpec((1, NPAD), lambda i: (0, 0)),
        ],
        out_specs=[
            pl.BlockSpec((1, BLK, 128), lambda i: (0, i, 0)),
            pl.BlockSpec((1, BLK), lambda i: (0, i)),
            pl.BlockSpec((1, 128), lambda i: (0, 0)),
        ],
        out_shape=[
            jax.ShapeDtypeStruct((1, N, 128), f32),
            jax.ShapeDtypeStruct((1, NPAD), f32),
            jax.ShapeDtypeStruct((1, 128), i32),
        ],
    )(x, W1, degp, batch_pad)

    hs1f = hs1.reshape(N, 128)
    acc1 = _mp1_kernel(edges, edges[0].reshape(CHUNKS, LANE), hs1f)

    hs2 = pl.pallas_call(
        _tc2_body,
        grid=(grid,),
        in_specs=[
            pl.BlockSpec((2, BLK, 128), lambda i: (0, i, 0)),
            pl.BlockSpec((1, BLK, 128), lambda i: (0, i, 0)),
            pl.BlockSpec((1, BLK), lambda i: (0, i)),
            pl.BlockSpec((1, 128), lambda i: (0, 0)),
            pl.BlockSpec((128, 256), lambda i: (0, 0)),
        ],
        out_specs=pl.BlockSpec((2, BLK, 128), lambda i: (0, i, 0)),
        out_shape=jax.ShapeDtypeStruct((2, N, 128), f32),
    )(acc1, hs1, dinv, b1.reshape(1, 128), W2)

    hs2f = hs2.reshape(2 * N, 128)
    acc2 = _mp2_kernel(edges, src_hi, hs2f)

    h2 = pl.pallas_call(
        _tc3_body,
        grid=(grid,),
        in_specs=[
            pl.BlockSpec((2, BLK, 128), lambda i: (0, i, 0)),
            pl.BlockSpec((2, BLK, 128), lambda i: (0, i, 0)),
            pl.BlockSpec((1, BLK), lambda i: (0, i)),
            pl.BlockSpec((2, 128), lambda i: (0, 0)),
        ],
        out_specs=pl.BlockSpec((BLK, 256), lambda i: (i, 0)),
        out_shape=jax.ShapeDtypeStruct((N, 256), f32),
    )(acc2, hs2, dinv, b2.reshape(2, 128))

    g = _pool_kernel(h2, bnds).reshape(G, 256)

    out = pl.pallas_call(
        _tc4_body,
        in_specs=[
            pl.BlockSpec((G, 256), lambda: (0, 0)),
            pl.BlockSpec((256, 512), lambda: (0, 0)),
            pl.BlockSpec((1, 512), lambda: (0, 0)),
            pl.BlockSpec((512, 128), lambda: (0, 0)),
            pl.BlockSpec((1, 128), lambda: (0, 0)),
        ],
        out_specs=pl.BlockSpec((G, 128), lambda: (0, 0)),
        out_shape=jax.ShapeDtypeStruct((G, 128), f32),
    )(g, fc1_W, fc1_b.reshape(1, 512), fc2_W, fc2_b.reshape(1, 128))
    return out


# speedup vs baseline: 2.2643x; 1.0073x over previous
import functools

import jax
import jax.numpy as jnp
from jax import lax
from jax.experimental import pallas as pl
from jax.experimental.pallas import tpu as pltpu
from jax.experimental.pallas import tpu_sc as plsc

N = 10000
E = 320000
G = 64
NPAD = 10240
LANE = 128
EPAD = 327680
CHUNKS = EPAD // LANE
ROWS_PER_TILE = NPAD // 16
BLK = 2048

f32 = jnp.float32
i32 = jnp.int32


def _zero_block(ref):
    def body(r, _):
        for k in range(8):
            ref[r, pl.ds(k * 16, 16)] = jnp.zeros((16,), f32)
        return 0
    lax.fori_loop(0, 128, body, 0)


def _deg_body(edges, degp, dstv, onesv, drainv, degsh):
    c = lax.axis_index("c")
    s = lax.axis_index("s")
    w = c * 16 + s
    nch = EPAD // 32 // LANE

    def init(k, _):
        drainv[pl.ds(k * 16, 16)] = jnp.zeros((16,), f32)
        return 0
    lax.fori_loop(0, ROWS_PER_TILE // 16, init, 0)
    for k in range(LANE // 16):
        onesv[pl.ds(k * 16, 16)] = jnp.ones((16,), f32)
    pltpu.sync_copy(drainv, degsh.at[pl.ds(s * ROWS_PER_TILE, ROWS_PER_TILE)])
    plsc.subcore_barrier()

    pltpu.sync_copy(edges.at[1, pl.ds(w * nch, nch)], dstv)

    def body(j, _):
        pltpu.sync_copy(onesv, degsh.at[dstv.at[j]], add=True)
        return 0
    lax.fori_loop(0, nch, body, 0)
    plsc.subcore_barrier()

    pltpu.sync_copy(degsh.at[pl.ds(s * ROWS_PER_TILE, ROWS_PER_TILE)], drainv)
    pltpu.sync_copy(
        drainv, degp.at[pl.ds(c * NPAD + s * ROWS_PER_TILE, ROWS_PER_TILE)])


STAGE = 40


def _mp_loop(copy_stage, table, srcv, dstv, rbuf, accsh, gs0, gs1, ss0, ss1,
             nstages):
    npairs = STAGE // 2

    def stage_body(st, _):
        copy_stage(st)
        pltpu.async_copy(table.at[srcv.at[0]], rbuf.at[0], gs0)
        pltpu.async_copy(table.at[srcv.at[1]], rbuf.at[1], gs1)

        def pair(j2, _):
            j0 = j2 * 2
            j1 = j0 + 1
            pltpu.make_async_copy(table.at[srcv.at[j0]], rbuf.at[0],
                                  gs0).wait()
            pltpu.async_copy(rbuf.at[0], accsh.at[dstv.at[j0]], ss0,
                             add=True)
            pltpu.make_async_copy(table.at[srcv.at[j1]], rbuf.at[1],
                                  gs1).wait()
            pltpu.async_copy(rbuf.at[1], accsh.at[dstv.at[j1]], ss1,
                             add=True)

            @pl.when(j2 < npairs - 1)
            def _():
                pltpu.make_async_copy(rbuf.at[0], accsh.at[dstv.at[j0]],
                                      ss0).wait()
                pltpu.async_copy(table.at[srcv.at[j0 + 2]], rbuf.at[0], gs0)
                pltpu.make_async_copy(rbuf.at[1], accsh.at[dstv.at[j1]],
                                      ss1).wait()
                pltpu.async_copy(table.at[srcv.at[j1 + 2]], rbuf.at[1], gs1)
            return 0
        lax.fori_loop(0, npairs, pair, 0)
        pltpu.make_async_copy(rbuf.at[0], accsh.at[dstv.at[0]], ss0).wait()
        pltpu.make_async_copy(rbuf.at[1], accsh.at[dstv.at[1]], ss1).wait()
        return 0
    lax.fori_loop(0, nstages, stage_body, 0)


def _mp_prologue(rbuf, accsh, sem, s):
    _zero_block(rbuf.at[0])
    nq = ROWS_PER_TILE // LANE
    descs = []
    for q in range(nq):
        descs.append(pltpu.async_copy(
            rbuf.at[0], accsh.at[pl.ds(s * ROWS_PER_TILE + q * LANE, LANE)],
            sem))
    for d in descs:
        d.wait()
    plsc.subcore_barrier()


def _mp_drain(rbuf, accsh, out, c, s, sem0, sem1):
    plsc.subcore_barrier()
    sems = (sem0, sem1)
    nq = ROWS_PER_TILE // LANE
    descs = [None, None]
    for q in range(nq):
        b = q % 2
        rows = pl.ds(s * ROWS_PER_TILE + q * LANE, LANE)
        if descs[b] is not None:
            descs[b].wait()
        pltpu.sync_copy(accsh.at[rows], rbuf.at[b])
        descs[b] = pltpu.async_copy(rbuf.at[b], out.at[c, rows], sems[b])
    for d in descs:
        if d is not None:
            d.wait()


def _make_mp_body(nch, per_core_edges):
    def body(edges, src_hi, table, acc, srcv, dstv, rbuf, accsh, gs0, gs1,
             ss0, ss1):
        c = lax.axis_index("c")
        s = lax.axis_index("s")
        base = ((c * 16 + s) if per_core_edges else s) * nch
        _mp_prologue(rbuf, accsh, gs0, s)

        def copy_stage(st):
            rows = pl.ds(base + st * STAGE, STAGE)

            @pl.when(c == 0)
            def _():
                pltpu.sync_copy(edges.at[0, rows], srcv)

            @pl.when(c == 1)
            def _():
                pltpu.sync_copy(src_hi.at[rows], srcv)

            pltpu.sync_copy(edges.at[1, rows], dstv)

        _mp_loop(copy_stage, table, srcv, dstv, rbuf, accsh, gs0, gs1,
                 ss0, ss1, nch // STAGE)
        _mp_drain(rbuf, accsh, acc, c, s, gs0, gs1)
    return body


_mp1_body = _make_mp_body(EPAD // 32 // LANE, True)
_mp2_body = _make_mp_body(EPAD // 16 // LANE, False)


def _pool_body(h2, bnds, gout, bndv, cbuf, maccv):
    c = lax.axis_index("c")
    s = lax.axis_index("s")
    t = c * 16 + s
    g0 = t * 2
    pltpu.sync_copy(bnds, bndv)
    bvec = bndv[0, pl.ds(g0, 16)]
    c0, c1, c2 = bvec[0], bvec[1], bvec[2]

    for gi in range(2):
        start = c0 if gi == 0 else c1
        end = c1 if gi == 0 else c2
        minit = tuple(jnp.full((16,), -jnp.inf, f32) for _ in range(16))
        nchunks = (end - start + 63) // 64

        def chunk_body(ch, m):
            pos = start + ch * 64
            cstart = jnp.minimum((pos // 8) * 8, N - 72)
            off = pos - cstart
            pltpu.sync_copy(h2.at[pl.ds(cstart, 72)], cbuf)
            nv = jnp.minimum(64, end - pos)

            def row_body(r, m):
                row = off + r
                return tuple(
                    jnp.maximum(m[cc], cbuf[row, pl.ds(cc * 16, 16)])
                    for cc in range(16))
            return lax.fori_loop(0, nv, row_body, m)

        m = lax.fori_loop(0, nchunks, chunk_body, minit)
        for cc in range(16):
            maccv[pl.ds(cc * 16, 16)] = m[cc]
        pltpu.sync_copy(maccv, gout.at[pl.ds((g0 + gi) * 256, 256)])


@functools.cache
def _sc_kernels():
    mesh = plsc.VectorSubcoreMesh(core_axis_name="c", subcore_axis_name="s")
    deg = pl.kernel(
        _deg_body,
        out_type=jax.ShapeDtypeStruct((2 * NPAD,), f32),
        scratch_types=[
            pltpu.VMEM((EPAD // 32 // LANE, LANE), i32),
            pltpu.VMEM((LANE,), f32),
            pltpu.VMEM((ROWS_PER_TILE,), f32),
            pltpu.VMEM_SHARED((NPAD,), f32),
        ],
        mesh=mesh,
    )
    mp_scratch = [
        pltpu.VMEM((STAGE, LANE), i32),
        pltpu.VMEM((STAGE, LANE), i32),
        pltpu.VMEM((2, LANE, 128), f32),
        pltpu.VMEM_SHARED((NPAD, 128), f32),
        pltpu.SemaphoreType.DMA,
        pltpu.SemaphoreType.DMA,
        pltpu.SemaphoreType.DMA,
        pltpu.SemaphoreType.DMA,
    ]
    mp1 = pl.kernel(
        _mp1_body,
        out_type=jax.ShapeDtypeStruct((2, NPAD, 128), f32),
        scratch_types=list(mp_scratch),
        mesh=mesh,
    )
    mp2 = pl.kernel(
        _mp2_body,
        out_type=jax.ShapeDtypeStruct((2, NPAD, 128), f32),
        scratch_types=list(mp_scratch),
        mesh=mesh,
    )
    pool = pl.kernel(
        _pool_body,
        out_type=jax.ShapeDtypeStruct((G * 256,), f32),
        scratch_types=[
            pltpu.VMEM((1, 128), i32),
            pltpu.VMEM((72, 256), f32),
            pltpu.VMEM((256,), f32),
        ],
        mesh=mesh,
    )
    return deg, mp1, mp2, pool


def _tc1_body(x_ref, w_ref, deg_ref, batch_ref, hs_ref, dinv_ref, bnd_ref):
    d = deg_ref[0, :] + deg_ref[1, :] + 1.0
    dinv = lax.rsqrt(d)
    dinv_ref[0, :] = dinv
    h = jnp.dot(x_ref[...], w_ref[...], preferred_element_type=f32)
    hs_ref[0] = h * dinv[:, None]

    @pl.when(pl.program_id(0) == 0)
    def _():
        b = batch_ref[0, :][:, None]
        gids = lax.broadcasted_iota(i32, (1, 128), 1)
        bnd_ref[...] = jnp.sum((b < gids).astype(i32), axis=0, keepdims=True)


def _tc2_body(acc_ref, hs1_ref, dinv_ref, b1_ref, w2_ref, out_ref):
    dinv = dinv_ref[0, :]
    z = dinv[:, None] * (acc_ref[0] + acc_ref[1] + hs1_ref[0])
    z = jnp.maximum(z + b1_ref[0, :][None, :], 0.0)
    h = jnp.dot(z, w2_ref[...], preferred_element_type=f32) * dinv[:, None]
    out_ref[0] = h[:, :128]
    out_ref[1] = h[:, 128:]


def _tc3_body(acc_ref, hs2_ref, dinv_ref, b2_ref, out_ref):
    dinv = dinv_ref[0, :]
    for half in range(2):
        v = dinv[:, None] * (acc_ref[half] + hs2_ref[half])
        v = jnp.maximum(v + b2_ref[half, :][None, :], 0.0)
        out_ref[:, half * 128:(half + 1) * 128] = v


def _tc4_body(g_ref, w1_ref, b1_ref, w2_ref, b2_ref, out_ref):
    t = jnp.dot(g_ref[...], w1_ref[...], preferred_element_type=f32)
    t = t + b1_ref[0, :][None, :]
    o = jnp.dot(t, w2_ref[...], preferred_element_type=f32)
    out_ref[...] = o + b2_ref[0, :][None, :]


def kernel(x, edge_index, batch, W1, b1, W2, b2, fc1_W, fc1_b, fc2_W, fc2_b):
    pad = EPAD - E
    src = jnp.concatenate(
        [edge_index[0], jnp.arange(pad, dtype=i32) % N])
    dst = jnp.concatenate(
        [edge_index[1], N + (jnp.arange(pad, dtype=i32) % (NPAD - N))])
    edges = jnp.stack([src, dst]).reshape(2, CHUNKS, LANE)
    src_hi = (src + N).reshape(CHUNKS, LANE)

    _deg_kernel, _mp1_kernel, _mp2_kernel, _pool_kernel = _sc_kernels()
    degp = _deg_kernel(edges).reshape(2, NPAD)

    batch_pad = jnp.concatenate(
        [batch, jnp.full((NPAD - N,), G, i32)]).reshape(1, NPAD)

    grid = NPAD // BLK
    hs1, dinv, bnds = pl.pallas_call(
        _tc1_body,
        grid=(grid,),
        in_specs=[
            pl.BlockSpec((BLK, 128), lambda i: (i, 0)),
            pl.BlockSpec((128, 128), lambda i: (0, 0)),
            pl.BlockSpec((2, BLK), lambda i: (0, i)),
            pl.BlockSpec((1, NPAD), lambda i: (0, 0)),
        ],
        out_specs=[
            pl.BlockSpec((1, BLK, 128), lambda i: (0, i, 0)),
            pl.BlockSpec((1, BLK), lambda i: (0, i)),
            pl.BlockSpec((1, 128), lambda i: (0, 0)),
        ],
        out_shape=[
            jax.ShapeDtypeStruct((1, N, 128), f32),
            jax.ShapeDtypeStruct((1, NPAD), f32),
            jax.ShapeDtypeStruct((1, 128), i32),
        ],
    )(x, W1, degp, batch_pad)

    hs1f = hs1.reshape(N, 128)
    acc1 = _mp1_kernel(edges, edges[0].reshape(CHUNKS, LANE), hs1f)

    hs2 = pl.pallas_call(
        _tc2_body,
        grid=(grid,),
        in_specs=[
            pl.BlockSpec((2, BLK, 128), lambda i: (0, i, 0)),
            pl.BlockSpec((1, BLK, 128), lambda i: (0, i, 0)),
            pl.BlockSpec((1, BLK), lambda i: (0, i)),
            pl.BlockSpec((1, 128), lambda i: (0, 0)),
            pl.BlockSpec((128, 256), lambda i: (0, 0)),
        ],
        out_specs=pl.BlockSpec((2, BLK, 128), lambda i: (0, i, 0)),
        out_shape=jax.ShapeDtypeStruct((2, N, 128), f32),
    )(acc1, hs1, dinv, b1.reshape(1, 128), W2)

    hs2f = hs2.reshape(2 * N, 128)
    acc2 = _mp2_kernel(edges, src_hi, hs2f)

    h2 = pl.pallas_call(
        _tc3_body,
        grid=(grid,),
        in_specs=[
            pl.BlockSpec((2, BLK, 128), lambda i: (0, i, 0)),
            pl.BlockSpec((2, BLK, 128), lambda i: (0, i, 0)),
            pl.BlockSpec((1, BLK), lambda i: (0, i)),
            pl.BlockSpec((2, 128), lambda i: (0, 0)),
        ],
        out_specs=pl.BlockSpec((BLK, 256), lambda i: (i, 0)),
        out_shape=jax.ShapeDtypeStruct((N, 256), f32),
    )(acc2, hs2, dinv, b2.reshape(2, 128))

    g = _pool_kernel(h2, bnds).reshape(G, 256)

    out = pl.pallas_call(
        _tc4_body,
        in_specs=[
            pl.BlockSpec((G, 256), lambda: (0, 0)),
            pl.BlockSpec((256, 512), lambda: (0, 0)),
            pl.BlockSpec((1, 512), lambda: (0, 0)),
            pl.BlockSpec((512, 128), lambda: (0, 0)),
            pl.BlockSpec((1, 128), lambda: (0, 0)),
        ],
        out_specs=pl.BlockSpec((G, 128), lambda: (0, 0)),
        out_shape=jax.ShapeDtypeStruct((G, 128), f32),
    )(g, fc1_W, fc1_b.reshape(1, 512), fc2_W, fc2_b.reshape(1, 128))
    return out
